# baseline (speedup 1.0000x reference)
import functools

import jax
import jax.numpy as jnp
from jax.experimental import pallas as pl
from jax.experimental.pallas import tpu as pltpu

B, N, H, K = 8, 1024, 64, 9
BLOCK_N = 128


def _body(idx_ref, m_ref, k_ref, v_ref, o_ref):
    b = pl.program_id(0)
    j = pl.program_id(1)
    o_ref[...] = m_ref[...]
    for kk in range(K):
        idx = idx_ref[b, kk]
        loc = idx - j * BLOCK_N
        inb = (loc >= 0) & (loc < BLOCK_N)
        safe = jnp.where(inb, loc, 0)
        krow = k_ref[0, safe, :]
        vrow = v_ref[0, safe, :]
        scale = jnp.where(inb, 1.0, 0.0).astype(jnp.float32)
        upd = (krow * scale)[:, None] * vrow[None, :]
        o_ref[0, safe, :, :] += upd


@jax.jit
def kernel(M, M_k, M_v, indices_update):
    idx = indices_update.astype(jnp.int32)
    grid = (B, N // BLOCK_N)
    return pl.pallas_call(
        _body,
        grid=grid,
        in_specs=[
            pl.BlockSpec(memory_space=pltpu.SMEM),
            pl.BlockSpec((1, BLOCK_N, H, H), lambda b, j: (b, j, 0, 0)),
            pl.BlockSpec((1, BLOCK_N, H), lambda b, j: (b, j, 0)),
            pl.BlockSpec((1, BLOCK_N, H), lambda b, j: (b, j, 0)),
        ],
        out_specs=pl.BlockSpec((1, BLOCK_N, H, H), lambda b, j: (b, j, 0, 0)),
        out_shape=jax.ShapeDtypeStruct((B, N, H, H), jnp.float32),
        compiler_params=pltpu.CompilerParams(
            dimension_semantics=("parallel", "parallel"),
        ),
    )(idx, M, M_k, M_v)


# speedup vs baseline: 1.2897x; 1.2897x over previous
import jax
import jax.numpy as jnp
from jax.experimental import pallas as pl
from jax.experimental.pallas import tpu as pltpu

B, N, H, K = 8, 1024, 64, 9


def _body(idx_ref, m_ref, k_ref, v_ref, o_ref):
    b = pl.program_id(0)
    k = pl.program_id(1)
    idx = idx_ref[b, k]
    prev = idx_ref[b, jnp.maximum(k - 1, 0)]
    first = (k == 0) | (idx != prev)
    krow = k_ref[0, 0, 0, :]
    vrow = v_ref[0, 0, 0, :]
    upd = krow[:, None] * vrow[None, :]

    @pl.when(first)
    def _():
        o_ref[0, 0] = m_ref[0, 0] + upd

    @pl.when(jnp.logical_not(first))
    def _():
        o_ref[0, 0] += upd


@jax.jit
def kernel(M, M_k, M_v, indices_update):
    idx = jnp.sort(indices_update.astype(jnp.int32), axis=1)
    grid_spec = pltpu.PrefetchScalarGridSpec(
        num_scalar_prefetch=1,
        grid=(B, K),
        in_specs=[
            pl.BlockSpec((1, 1, H, H), lambda b, k, i: (b, i[b, k], 0, 0)),
            pl.BlockSpec((1, 1, 1, H), lambda b, k, i: (b, i[b, k], 0, 0)),
            pl.BlockSpec((1, 1, 1, H), lambda b, k, i: (b, i[b, k], 0, 0)),
        ],
        out_specs=pl.BlockSpec((1, 1, H, H), lambda b, k, i: (b, i[b, k], 0, 0)),
    )
    return pl.pallas_call(
        _body,
        grid_spec=grid_spec,
        out_shape=jax.ShapeDtypeStruct((B, N, H, H), jnp.float32),
        input_output_aliases={1: 0},
        compiler_params=pltpu.CompilerParams(
            dimension_semantics=("arbitrary", "arbitrary"),
        ),
    )(idx, M, M_k[:, :, None, :], M_v[:, :, None, :])


# speedup vs baseline: 1.3773x; 1.0679x over previous
import jax
import jax.numpy as jnp
from jax.experimental import pallas as pl
from jax.experimental.pallas import tpu as pltpu

B, N, H, K = 8, 1024, 64, 9


def _body(idx_ref, k_ref, v_ref, m_ref, o_ref, buf, in_sems, out_sems):
    flat = lambda b, k: b * K + k

    def slot_info(b, k):
        idx = idx_ref[b, k]
        mult = 1
        first = True
        for j in range(K):
            if j == k:
                continue
            same = idx_ref[b, j] == idx
            mult = mult + same.astype(jnp.int32)
            if j < k:
                first = jnp.logical_and(first, jnp.logical_not(same))
        return idx, mult, first

    for b in range(B):
        for k in range(K):
            idx, mult, first = slot_info(b, k)
            bk = flat(b, k)

            @pl.when(first)
            def _(idx=idx, bk=bk):
                pltpu.make_async_copy(
                    o_ref.at[b, idx], buf.at[bk], in_sems.at[bk]
                ).start()

    for b in range(B):
        for k in range(K):
            idx, mult, first = slot_info(b, k)
            bk = flat(b, k)

            @pl.when(first)
            def _(idx=idx, mult=mult, bk=bk, b=b, k=k):
                pltpu.make_async_copy(
                    o_ref.at[b, idx], buf.at[bk], in_sems.at[bk]
                ).wait()
                krow = k_ref[b, idx, :] * mult.astype(jnp.float32)
                vrow = v_ref[b, idx, :]
                buf[bk] += krow[:, None] * vrow[None, :]
                pltpu.make_async_copy(
                    buf.at[bk], o_ref.at[b, idx], out_sems.at[bk]
                ).start()

    for b in range(B):
        for k in range(K):
            idx, mult, first = slot_info(b, k)
            bk = flat(b, k)

            @pl.when(first)
            def _(idx=idx, bk=bk, b=b):
                pltpu.make_async_copy(
                    buf.at[bk], o_ref.at[b, idx], out_sems.at[bk]
                ).wait()


@jax.jit
def kernel(M, M_k, M_v, indices_update):
    idx = indices_update.astype(jnp.int32)
    return pl.pallas_call(
        _body,
        in_specs=[
            pl.BlockSpec(memory_space=pltpu.SMEM),
            pl.BlockSpec(memory_space=pltpu.VMEM),
            pl.BlockSpec(memory_space=pltpu.VMEM),
            pl.BlockSpec(memory_space=pl.ANY),
        ],
        out_specs=pl.BlockSpec(memory_space=pl.ANY),
        out_shape=jax.ShapeDtypeStruct((B, N, H, H), jnp.float32),
        input_output_aliases={3: 0},
        scratch_shapes=[
            pltpu.VMEM((B * K, H, H), jnp.float32),
            pltpu.SemaphoreType.DMA((B * K,)),
            pltpu.SemaphoreType.DMA((B * K,)),
        ],
    )(idx, M_k, M_v, M)
